# baseline (speedup 1.0000x reference)
import jax
import jax.numpy as jnp
from jax import lax
from jax.experimental import pallas as pl
from jax.experimental.pallas import tpu as pltpu

_N = 4096
_K = 30
_R = 128
_F32 = jnp.float32
_BF16 = jnp.bfloat16
_W = (0.28, 0.2, 0.17, 0.5)


def _mm_nt(a, b):
    return lax.dot_general(a.astype(_BF16), b.astype(_BF16),
                           (((1,), (1,)), ((), ())),
                           preferred_element_type=_F32)


def _mm_nn(a, b):
    return lax.dot_general(a.astype(_BF16), b.astype(_BF16),
                           (((1,), (0,)), ((), ())),
                           preferred_element_type=_F32)


def _body(x_ref, xo_ref, wp_ref, wpo_ref, bp_ref, bpo_ref,
          out_ref, scores_ref,
          xn_ref, xno_ref, xnt_ref, xnot_ref):
    i = pl.program_id(0)

    @pl.when(i == 0)
    def _prep():
        xp = _mm_nt(x_ref[...], wp_ref[...]) + bp_ref[...]
        n = jnp.maximum(jnp.sqrt(jnp.sum(xp * xp, axis=1, keepdims=True)), 1e-12)
        xn = xp / n
        xq = _mm_nt(xo_ref[...], wpo_ref[...]) + bpo_ref[...]
        no = jnp.maximum(jnp.sqrt(jnp.sum(xq * xq, axis=1, keepdims=True)), 1e-12)
        xno = xq / no
        xn_ref[...] = xn
        xno_ref[...] = xno
        xnt_ref[...] = xn.T
        xnot_ref[...] = xno.T

    xn_blk = xn_ref[pl.ds(i * _R, _R), :]
    xno_blk = xno_ref[pl.ds(i * _R, _R), :]
    s1 = _mm_nn(xn_blk, xnt_ref[...])
    xnot = xnot_ref[...]
    so = _W[0] * _mm_nn(xno_blk[:, 0:32], xnot[0:32, :])
    so = so + _W[1] * _mm_nn(xno_blk[:, 32:64], xnot[32:64, :])
    so = so + _W[2] * _mm_nn(xno_blk[:, 64:96], xnot[64:96, :])
    so = so + _W[3] * _mm_nn(xno_blk[:, 96:128], xnot[96:128, :])
    s = 0.9 * s1 + 0.1 * so
    scores_ref[...] = s

    bits = lax.bitcast_convert_type(s, jnp.int32)
    y = bits ^ (jnp.right_shift(bits, 31) & jnp.int32(0x7FFFFFFF))

    lo = jnp.min(y, axis=1, keepdims=True)
    hi = jnp.max(y, axis=1, keepdims=True)

    def body(_, carry):
        lo, hi = carry
        mid = lo + jnp.right_shift(hi - lo + 1, 1)
        cnt = jnp.sum((y >= mid).astype(jnp.int32), axis=1, keepdims=True)
        ok = cnt >= _K
        return jnp.where(ok, mid, lo), jnp.where(ok, hi, mid - 1)

    lo, _ = lax.fori_loop(0, 31, body, (lo, hi))
    out_ref[...] = jnp.where(y >= lo, s, 0.0)


def kernel(x, x_origin, Wp, bp, Wpo, bpo):
    bp2 = bp.reshape(1, -1)
    bpo2 = bpo.reshape(1, -1)

    nblk = _N // _R
    out, scores = pl.pallas_call(
        _body,
        grid=(nblk,),
        in_specs=[pl.BlockSpec((_N, 128), lambda i: (0, 0)),
                  pl.BlockSpec((_N, 128), lambda i: (0, 0)),
                  pl.BlockSpec((128, 128), lambda i: (0, 0)),
                  pl.BlockSpec((128, 128), lambda i: (0, 0)),
                  pl.BlockSpec((1, 128), lambda i: (0, 0)),
                  pl.BlockSpec((1, 128), lambda i: (0, 0))],
        out_specs=[pl.BlockSpec((_R, _N), lambda i: (i, 0)),
                   pl.BlockSpec((_R, _N), lambda i: (i, 0))],
        out_shape=[jax.ShapeDtypeStruct((_N, _N), _F32),
                   jax.ShapeDtypeStruct((_N, _N), _F32)],
        scratch_shapes=[pltpu.VMEM((_N, 128), _F32),
                        pltpu.VMEM((_N, 128), _F32),
                        pltpu.VMEM((128, _N), _F32),
                        pltpu.VMEM((128, _N), _F32)],
    )(x, x_origin, Wp, Wpo, bp2, bpo2)
    return (out, scores)


# speedup vs baseline: 1.1351x; 1.1351x over previous
import jax
import jax.numpy as jnp
from jax import lax
from jax.experimental import pallas as pl
from jax.experimental.pallas import tpu as pltpu

_N = 4096
_K = 30
_R = 256
_F32 = jnp.float32
_BF16 = jnp.bfloat16
_W = (0.28, 0.2, 0.17, 0.5)


def _mm_nt(a, b):
    return lax.dot_general(a.astype(_BF16), b.astype(_BF16),
                           (((1,), (1,)), ((), ())),
                           preferred_element_type=_F32)


def _mm_nn(a, b):
    return lax.dot_general(a.astype(_BF16), b.astype(_BF16),
                           (((1,), (0,)), ((), ())),
                           preferred_element_type=_F32)


def _body(x_ref, xo_ref, wp_ref, wpo_ref, bp_ref, bpo_ref,
          out_ref, scores_ref,
          xn_ref, xno_ref, xnt_ref, xnot_ref):
    i = pl.program_id(0)

    @pl.when(i == 0)
    def _prep():
        xp = _mm_nt(x_ref[...], wp_ref[...]) + bp_ref[...]
        n = jnp.maximum(jnp.sqrt(jnp.sum(xp * xp, axis=1, keepdims=True)), 1e-12)
        xn = xp / n
        xq = _mm_nt(xo_ref[...], wpo_ref[...]) + bpo_ref[...]
        no = jnp.maximum(jnp.sqrt(jnp.sum(xq * xq, axis=1, keepdims=True)), 1e-12)
        xno = xq / no
        xn_ref[...] = xn
        xno_ref[...] = xno
        xnt_ref[...] = xn.T
        xnot_ref[...] = xno.T

    xn_blk = xn_ref[pl.ds(i * _R, _R), :]
    xno_blk = xno_ref[pl.ds(i * _R, _R), :]
    s1 = _mm_nn(xn_blk, xnt_ref[...])
    xnot = xnot_ref[...]
    so = _W[0] * _mm_nn(xno_blk[:, 0:32], xnot[0:32, :])
    so = so + _W[1] * _mm_nn(xno_blk[:, 32:64], xnot[32:64, :])
    so = so + _W[2] * _mm_nn(xno_blk[:, 64:96], xnot[64:96, :])
    so = so + _W[3] * _mm_nn(xno_blk[:, 96:128], xnot[96:128, :])
    s = 0.9 * s1 + 0.1 * so
    scores_ref[...] = s

    bits = lax.bitcast_convert_type(s, jnp.int32)
    y = bits ^ (jnp.right_shift(bits, 31) & jnp.int32(0x7FFFFFFF))

    lo = jnp.min(y, axis=1, keepdims=True)
    hi = jnp.max(y, axis=1, keepdims=True)

    def body(_, carry):
        lo, hi = carry
        mid = lo + jnp.right_shift(hi - lo + 1, 1)
        cnt = jnp.sum((y >= mid).astype(jnp.int32), axis=1, keepdims=True)
        ok = cnt >= _K
        return jnp.where(ok, mid, lo), jnp.where(ok, hi, mid - 1)

    lo, _ = lax.fori_loop(0, 31, body, (lo, hi))
    out_ref[...] = jnp.where(y >= lo, s, 0.0)


def kernel(x, x_origin, Wp, bp, Wpo, bpo):
    bp2 = bp.reshape(1, -1)
    bpo2 = bpo.reshape(1, -1)

    nblk = _N // _R
    out, scores = pl.pallas_call(
        _body,
        grid=(nblk,),
        in_specs=[pl.BlockSpec((_N, 128), lambda i: (0, 0)),
                  pl.BlockSpec((_N, 128), lambda i: (0, 0)),
                  pl.BlockSpec((128, 128), lambda i: (0, 0)),
                  pl.BlockSpec((128, 128), lambda i: (0, 0)),
                  pl.BlockSpec((1, 128), lambda i: (0, 0)),
                  pl.BlockSpec((1, 128), lambda i: (0, 0))],
        out_specs=[pl.BlockSpec((_R, _N), lambda i: (i, 0)),
                   pl.BlockSpec((_R, _N), lambda i: (i, 0))],
        out_shape=[jax.ShapeDtypeStruct((_N, _N), _F32),
                   jax.ShapeDtypeStruct((_N, _N), _F32)],
        scratch_shapes=[pltpu.VMEM((_N, 128), _F32),
                        pltpu.VMEM((_N, 128), _F32),
                        pltpu.VMEM((128, _N), _F32),
                        pltpu.VMEM((128, _N), _F32)],
    )(x, x_origin, Wp, Wpo, bp2, bpo2)
    return (out, scores)


# speedup vs baseline: 1.1559x; 1.0183x over previous
import jax
import jax.numpy as jnp
from jax import lax
from jax.experimental import pallas as pl
from jax.experimental.pallas import tpu as pltpu

_N = 4096
_K = 30
_R = 256
_F32 = jnp.float32
_BF16 = jnp.bfloat16
_W = (0.28, 0.2, 0.17, 0.5)


def _mm_nt(a, b):
    return lax.dot_general(a.astype(_BF16), b.astype(_BF16),
                           (((1,), (1,)), ((), ())),
                           preferred_element_type=_F32)


def _mm_nn(a, b):
    return lax.dot_general(a.astype(_BF16), b.astype(_BF16),
                           (((1,), (0,)), ((), ())),
                           preferred_element_type=_F32)


def _body(x_ref, xo_ref, wp_ref, wpo_ref, bp_ref, bpo_ref,
          out_ref, scores_ref,
          xn_ref, xno_ref, xnt_ref, xnot_ref):
    i = pl.program_id(0)

    @pl.when(i == 0)
    def _prep():
        xp = _mm_nt(x_ref[...], wp_ref[...]) + bp_ref[...]
        n = jnp.maximum(jnp.sqrt(jnp.sum(xp * xp, axis=1, keepdims=True)), 1e-12)
        xn = xp / n
        xq = _mm_nt(xo_ref[...], wpo_ref[...]) + bpo_ref[...]
        no = jnp.maximum(jnp.sqrt(jnp.sum(xq * xq, axis=1, keepdims=True)), 1e-12)
        xno = xq / no
        xn_ref[...] = xn
        xno_ref[...] = xno
        xnt_ref[...] = xn.T
        xnot_ref[...] = xno.T

    xn_blk = xn_ref[pl.ds(i * _R, _R), :]
    xno_blk = xno_ref[pl.ds(i * _R, _R), :]
    s1 = _mm_nn(xn_blk, xnt_ref[...])
    xnot = xnot_ref[...]
    so = _W[0] * _mm_nn(xno_blk[:, 0:32], xnot[0:32, :])
    so = so + _W[1] * _mm_nn(xno_blk[:, 32:64], xnot[32:64, :])
    so = so + _W[2] * _mm_nn(xno_blk[:, 64:96], xnot[64:96, :])
    so = so + _W[3] * _mm_nn(xno_blk[:, 96:128], xnot[96:128, :])
    s = 0.9 * s1 + 0.1 * so
    scores_ref[...] = s

    bits = lax.bitcast_convert_type(s, jnp.int32)
    y = bits ^ (jnp.right_shift(bits, 31) & jnp.int32(0x7FFFFFFF))

    c_bound = jnp.int32(1066099302)
    lo = jnp.full((_R, 1), -c_bound, jnp.int32)
    hi = jnp.full((_R, 1), c_bound, jnp.int32)

    def body(_, carry):
        lo, hi = carry
        mid = lo + jnp.right_shift(hi - lo + 1, 1)
        cnt = jnp.sum((y >= mid).astype(jnp.int32), axis=1, keepdims=True)
        ok = cnt >= _K
        return jnp.where(ok, mid, lo), jnp.where(ok, hi, mid - 1)

    lo, _ = lax.fori_loop(0, 31, body, (lo, hi))
    out_ref[...] = jnp.where(y >= lo, s, 0.0)


def kernel(x, x_origin, Wp, bp, Wpo, bpo):
    bp2 = bp.reshape(1, -1)
    bpo2 = bpo.reshape(1, -1)

    nblk = _N // _R
    out, scores = pl.pallas_call(
        _body,
        grid=(nblk,),
        in_specs=[pl.BlockSpec((_N, 128), lambda i: (0, 0)),
                  pl.BlockSpec((_N, 128), lambda i: (0, 0)),
                  pl.BlockSpec((128, 128), lambda i: (0, 0)),
                  pl.BlockSpec((128, 128), lambda i: (0, 0)),
                  pl.BlockSpec((1, 128), lambda i: (0, 0)),
                  pl.BlockSpec((1, 128), lambda i: (0, 0))],
        out_specs=[pl.BlockSpec((_R, _N), lambda i: (i, 0)),
                   pl.BlockSpec((_R, _N), lambda i: (i, 0))],
        out_shape=[jax.ShapeDtypeStruct((_N, _N), _F32),
                   jax.ShapeDtypeStruct((_N, _N), _F32)],
        scratch_shapes=[pltpu.VMEM((_N, 128), _F32),
                        pltpu.VMEM((_N, 128), _F32),
                        pltpu.VMEM((128, _N), _F32),
                        pltpu.VMEM((128, _N), _F32)],
    )(x, x_origin, Wp, Wpo, bp2, bpo2)
    return (out, scores)
